# Initial kernel scaffold; baseline (speedup 1.0000x reference)
import jax
import jax.numpy as jnp
from jax.experimental import pallas as pl


def kernel(x, edge_index, W1, b1, W2, b2, W3, b3):
    raise NotImplementedError("write your pallas kernel here")


# speedup vs baseline: 6.0520x; 6.0520x over previous
import functools

import jax
import jax.numpy as jnp
from jax import lax
from jax.experimental import pallas as pl
from jax.experimental.pallas import tpu as pltpu
from jax.experimental.pallas import tpu_sc as plsc

_NC = 2
_NS = 16
_NW = _NC * _NS
_B = 128

_mesh = plsc.VectorSubcoreMesh(
    core_axis_name="c", subcore_axis_name="s", num_cores=_NC, num_subcores=_NS
)


def _make_sc_degree(n_pad, eb_rows, width):
  rows_per_sub = n_pad // _NS
  rows_per_w = eb_rows // _NW

  @functools.partial(
      pl.kernel,
      mesh=_mesh,
      out_type=jax.ShapeDtypeStruct((_NC * n_pad, width), jnp.float32),
      scratch_types=[
          pltpu.VMEM((rows_per_w, _B), jnp.int32),
          pltpu.VMEM((_B, width), jnp.float32),
          pltpu.VMEM_SHARED((n_pad, width), jnp.float32),
      ],
  )
  def deg_kernel(dst_hbm, zeros_hbm, ones_hbm, out_hbm, dst_v, ones_v, deg_sh):
    cid = lax.axis_index("c")
    sid = lax.axis_index("s")
    wid = sid * _NC + cid
    pltpu.sync_copy(
        zeros_hbm.at[pl.ds(sid * rows_per_sub, rows_per_sub)],
        deg_sh.at[pl.ds(sid * rows_per_sub, rows_per_sub)],
    )
    pltpu.sync_copy(ones_hbm, ones_v)
    pltpu.sync_copy(dst_hbm.at[pl.ds(wid * rows_per_w, rows_per_w)], dst_v)
    plsc.subcore_barrier()

    def body(j, carry):
      pltpu.sync_copy(ones_v, deg_sh.at[dst_v.at[j]], add=True)
      return carry

    lax.fori_loop(0, rows_per_w, body, 0)
    plsc.subcore_barrier()
    pltpu.sync_copy(
        deg_sh.at[pl.ds(sid * rows_per_sub, rows_per_sub)],
        out_hbm.at[pl.ds(cid * n_pad + sid * rows_per_sub, rows_per_sub)],
    )

  return deg_kernel


def _make_sc_aggregate(n, n_pad, eb_rows, h_half):
  rows_per_sub = (n // _NS) // 8 * 8
  tail = n - rows_per_sub * _NS
  rows_per_w = eb_rows // _NS

  @functools.partial(
      pl.kernel,
      mesh=_mesh,
      out_type=jax.ShapeDtypeStruct((_NC * n, h_half), jnp.float32),
      scratch_types=[
          pltpu.VMEM((rows_per_w, _B), jnp.int32),
          pltpu.VMEM((rows_per_w, _B), jnp.int32),
          pltpu.VMEM((_B, h_half), jnp.float32),
          pltpu.VMEM_SHARED((n_pad, h_half), jnp.float32),
          pltpu.SemaphoreType.DMA,
      ],
  )
  def agg_kernel(g_hbm, src_hbm, dst_hbm, out_hbm, src_v, dst_v, rows_v,
                 acc_sh, sem):
    cid = lax.axis_index("c")
    sid = lax.axis_index("s")
    pltpu.sync_copy(
        g_hbm.at[pl.ds(cid * n + sid * rows_per_sub, rows_per_sub)],
        acc_sh.at[pl.ds(sid * rows_per_sub, rows_per_sub)],
    )
    if tail:
      @pl.when(sid == 0)
      def _():
        pltpu.sync_copy(
            g_hbm.at[pl.ds(cid * n + rows_per_sub * _NS, tail)],
            acc_sh.at[pl.ds(rows_per_sub * _NS, tail)],
        )
    pltpu.sync_copy(src_hbm.at[pl.ds(sid * rows_per_w, rows_per_w)], src_v)
    pltpu.sync_copy(dst_hbm.at[pl.ds(sid * rows_per_w, rows_per_w)], dst_v)

    off = cid * n

    def obody(k, carry):
      j = k // (_B // 16)
      t = k % (_B // 16)
      sl = src_v[j, pl.ds(t * 16, 16)]
      src_v[j, pl.ds(t * 16, 16)] = sl + off
      return carry

    lax.fori_loop(0, rows_per_w * (_B // 16), obody, 0)
    plsc.subcore_barrier()

    def ebody(j, carry):
      pltpu.async_copy(g_hbm.at[src_v.at[j]], rows_v, sem).wait()
      pltpu.sync_copy(rows_v, acc_sh.at[dst_v.at[j]], add=True)
      return carry

    lax.fori_loop(0, rows_per_w, ebody, 0)
    plsc.subcore_barrier()
    pltpu.sync_copy(
        acc_sh.at[pl.ds(sid * rows_per_sub, rows_per_sub)],
        out_hbm.at[pl.ds(cid * n + sid * rows_per_sub, rows_per_sub)],
    )
    if tail:
      @pl.when(sid == 0)
      def _():
        pltpu.sync_copy(
            acc_sh.at[pl.ds(rows_per_sub * _NS, tail)],
            out_hbm.at[pl.ds(cid * n + rows_per_sub * _NS, tail)],
        )

  return agg_kernel


def _tc0_body(x_ref, w_ref, deg_ref, g_ref, dinv_ref):
  dinv = lax.rsqrt(deg_ref[:, 0:1] + 1.0)
  g_ref[...] = dinv * jnp.dot(
      x_ref[...], w_ref[...], preferred_element_type=jnp.float32
  )
  dinv_ref[...] = jnp.broadcast_to(dinv, dinv_ref.shape)


def _tc_layer_body(accl_ref, accr_ref, dinv_ref, b_ref, w_ref, g_ref):
  dinv = dinv_ref[:, 0:1]
  t = jnp.concatenate([accl_ref[...], accr_ref[...]], axis=1)
  t = jnp.maximum(dinv * t + b_ref[...], 0.0)
  g_ref[...] = dinv * jnp.dot(
      t, w_ref[...], preferred_element_type=jnp.float32
  )


def _tc_final_body(accl_ref, accr_ref, dinv_ref, b_ref, out_ref):
  dinv = dinv_ref[:, 0:1]
  t = dinv * jnp.concatenate([accl_ref[...], accr_ref[...]], axis=1) + b_ref[...]
  m = jnp.max(t, axis=1, keepdims=True)
  u = t - m
  out_ref[...] = u - jnp.log(jnp.sum(jnp.exp(u), axis=1, keepdims=True))


def kernel(x, edge_index, W1, b1, W2, b2, W3, b3):
  n, d = x.shape
  h = W1.shape[1]
  e = edge_index.shape[1]
  hh = h // 2
  rb = 1000
  ng = n // rb
  n_pad = ((n + 16 + 127) // 128) * 128
  e_pad = ((e + _NW * _B - 1) // (_NW * _B)) * (_NW * _B)
  eb_rows = e_pad // _B

  src = edge_index[0]
  dst = edge_index[1]
  pad = e_pad - e
  src2 = jnp.concatenate([src, jnp.zeros((pad,), jnp.int32)]).reshape(eb_rows, _B)
  dst2 = jnp.concatenate(
      [dst, jnp.full((pad,), n, jnp.int32)]
  ).reshape(eb_rows, _B)

  zeros128 = jnp.zeros((n_pad, 128), jnp.float32)
  ones128 = jnp.ones((_B, 128), jnp.float32)

  sc_deg = _make_sc_degree(n_pad, eb_rows, 128)
  sc_agg = _make_sc_aggregate(n, n_pad, eb_rows, hh)

  degp = sc_deg(dst2, zeros128, ones128)
  degsum = (degp[:n_pad, :8] + degp[n_pad:, :8])[:n]

  tc0 = pl.pallas_call(
      _tc0_body,
      grid=(ng, _NC),
      in_specs=[
          pl.BlockSpec((rb, d), lambda i, c: (i, 0)),
          pl.BlockSpec((d, hh), lambda i, c: (0, c)),
          pl.BlockSpec((rb, 8), lambda i, c: (i, 0)),
      ],
      out_specs=[
          pl.BlockSpec((rb, hh), lambda i, c: (c * ng + i, 0)),
          pl.BlockSpec((rb, 8), lambda i, c: (i, 0)),
      ],
      out_shape=[
          jax.ShapeDtypeStruct((_NC * n, hh), jnp.float32),
          jax.ShapeDtypeStruct((n, 8), jnp.float32),
      ],
  )
  g1, dinv8 = tc0(x, W1, degsum)

  def tc_layer(acc, w, b):
    return pl.pallas_call(
        _tc_layer_body,
        grid=(ng, _NC),
        in_specs=[
            pl.BlockSpec((rb, hh), lambda i, c: (i, 0)),
            pl.BlockSpec((rb, hh), lambda i, c: (ng + i, 0)),
            pl.BlockSpec((rb, 8), lambda i, c: (i, 0)),
            pl.BlockSpec((1, h), lambda i, c: (0, 0)),
            pl.BlockSpec((h, hh), lambda i, c: (0, c)),
        ],
        out_specs=pl.BlockSpec((rb, hh), lambda i, c: (c * ng + i, 0)),
        out_shape=jax.ShapeDtypeStruct((_NC * n, hh), jnp.float32),
    )(acc, acc, dinv8, b.reshape(1, h), w)

  acc1 = sc_agg(g1, src2, dst2)
  g2 = tc_layer(acc1, W2, b1)
  acc2 = sc_agg(g2, src2, dst2)
  g3 = tc_layer(acc2, W3, b2)
  acc3 = sc_agg(g3, src2, dst2)

  out = pl.pallas_call(
      _tc_final_body,
      grid=(ng,),
      in_specs=[
          pl.BlockSpec((rb, hh), lambda i: (i, 0)),
          pl.BlockSpec((rb, hh), lambda i: (ng + i, 0)),
          pl.BlockSpec((rb, 8), lambda i: (i, 0)),
          pl.BlockSpec((1, h), lambda i: (0, 0)),
      ],
      out_specs=pl.BlockSpec((rb, h), lambda i: (i, 0)),
      out_shape=jax.ShapeDtypeStruct((n, h), jnp.float32),
  )(acc3, acc3, dinv8, b3.reshape(1, h))
  return out


# speedup vs baseline: 7.0609x; 1.1667x over previous
import functools

import jax
import jax.numpy as jnp
from jax import lax
from jax.experimental import pallas as pl
from jax.experimental.pallas import tpu as pltpu
from jax.experimental.pallas import tpu_sc as plsc

_NC = 2
_NS = 16
_NW = _NC * _NS
_B = 128

_mesh = plsc.VectorSubcoreMesh(
    core_axis_name="c", subcore_axis_name="s", num_cores=_NC, num_subcores=_NS
)


def _make_sc_degree(n_pad, eb_rows, width):
  rows_per_sub = n_pad // _NS
  rows_per_w = eb_rows // _NW

  @functools.partial(
      pl.kernel,
      mesh=_mesh,
      out_type=jax.ShapeDtypeStruct((_NC * n_pad, width), jnp.float32),
      scratch_types=[
          pltpu.VMEM((rows_per_w, _B), jnp.int32),
          pltpu.VMEM((_B, width), jnp.float32),
          pltpu.VMEM_SHARED((n_pad, width), jnp.float32),
      ],
  )
  def deg_kernel(dst_hbm, zeros_hbm, ones_hbm, out_hbm, dst_v, ones_v, deg_sh):
    cid = lax.axis_index("c")
    sid = lax.axis_index("s")
    wid = sid * _NC + cid
    pltpu.sync_copy(
        zeros_hbm.at[pl.ds(sid * rows_per_sub, rows_per_sub)],
        deg_sh.at[pl.ds(sid * rows_per_sub, rows_per_sub)],
    )
    pltpu.sync_copy(ones_hbm, ones_v)
    pltpu.sync_copy(dst_hbm.at[pl.ds(wid * rows_per_w, rows_per_w)], dst_v)
    plsc.subcore_barrier()

    def body(j, carry):
      pltpu.sync_copy(ones_v, deg_sh.at[dst_v.at[j]], add=True)
      return carry

    lax.fori_loop(0, rows_per_w, body, 0)
    plsc.subcore_barrier()
    pltpu.sync_copy(
        deg_sh.at[pl.ds(sid * rows_per_sub, rows_per_sub)],
        out_hbm.at[pl.ds(cid * n_pad + sid * rows_per_sub, rows_per_sub)],
    )

  return deg_kernel


def _make_sc_aggregate(n, n_pad, eb_rows, h_half):
  rows_per_sub = (n // _NS) // 8 * 8
  tail = n - rows_per_sub * _NS
  rows_per_w = eb_rows // _NS

  @functools.partial(
      pl.kernel,
      mesh=_mesh,
      out_type=jax.ShapeDtypeStruct((_NC * n, h_half), jnp.float32),
      scratch_types=[
          pltpu.VMEM((rows_per_w // 2, _B), jnp.int32),
          pltpu.VMEM((rows_per_w // 2, _B), jnp.int32),
          pltpu.VMEM((_B, h_half), jnp.float32),
          pltpu.VMEM((_B, h_half), jnp.float32),
          pltpu.VMEM_SHARED((n_pad, h_half), jnp.float32),
          pltpu.SemaphoreType.DMA,
      ],
  )
  def agg_kernel(g_hbm, src_hbm, dst_hbm, out_hbm, src_v, dst_v, rows_a,
                 rows_b, acc_sh, sem_a):
    cid = lax.axis_index("c")
    sid = lax.axis_index("s")
    pltpu.sync_copy(
        g_hbm.at[pl.ds(cid * n + sid * rows_per_sub, rows_per_sub)],
        acc_sh.at[pl.ds(sid * rows_per_sub, rows_per_sub)],
    )
    if tail:
      @pl.when(sid == 0)
      def _():
        pltpu.sync_copy(
            g_hbm.at[pl.ds(cid * n + rows_per_sub * _NS, tail)],
            acc_sh.at[pl.ds(rows_per_sub * _NS, tail)],
        )
    plsc.subcore_barrier()

    half_rows = rows_per_w // 2
    n_pairs = half_rows // 2
    off = cid * n

    for chunk in range(2):
      base = sid * rows_per_w + chunk * half_rows
      pltpu.sync_copy(src_hbm.at[pl.ds(base, half_rows)], src_v)
      pltpu.sync_copy(dst_hbm.at[pl.ds(base, half_rows)], dst_v)

      def obody(k, carry):
        j = k // (_B // 16)
        t = k % (_B // 16)
        sl = src_v[j, pl.ds(t * 16, 16)]
        src_v[j, pl.ds(t * 16, 16)] = sl + off
        return carry

      lax.fori_loop(0, half_rows * (_B // 16), obody, 0)

      pltpu.async_copy(g_hbm.at[src_v.at[0]], rows_a, sem_a)

      def ebody(i, carry):
        j = 2 * i
        pltpu.async_copy(g_hbm.at[src_v.at[j + 1]], rows_b, sem_a)
        pltpu.make_async_copy(g_hbm.at[src_v.at[j]], rows_a, sem_a).wait()
        pltpu.sync_copy(rows_a, acc_sh.at[dst_v.at[j]], add=True)

        @pl.when(i < n_pairs - 1)
        def _():
          pltpu.async_copy(g_hbm.at[src_v.at[j + 2]], rows_a, sem_a)

        pltpu.make_async_copy(g_hbm.at[src_v.at[j + 1]], rows_b, sem_a).wait()
        pltpu.sync_copy(rows_b, acc_sh.at[dst_v.at[j + 1]], add=True)
        return carry

      lax.fori_loop(0, n_pairs, ebody, 0)
    plsc.subcore_barrier()
    pltpu.sync_copy(
        acc_sh.at[pl.ds(sid * rows_per_sub, rows_per_sub)],
        out_hbm.at[pl.ds(cid * n + sid * rows_per_sub, rows_per_sub)],
    )
    if tail:
      @pl.when(sid == 0)
      def _():
        pltpu.sync_copy(
            acc_sh.at[pl.ds(rows_per_sub * _NS, tail)],
            out_hbm.at[pl.ds(cid * n + rows_per_sub * _NS, tail)],
        )

  return agg_kernel


def _tc0_body(x_ref, w_ref, deg_ref, g_ref, dinv_ref):
  dinv = lax.rsqrt(deg_ref[:, 0:1] + 1.0)
  g_ref[...] = dinv * jnp.dot(
      x_ref[...], w_ref[...], preferred_element_type=jnp.float32
  )
  dinv_ref[...] = jnp.broadcast_to(dinv, dinv_ref.shape)


def _tc_layer_body(accl_ref, accr_ref, dinv_ref, b_ref, w_ref, g_ref):
  dinv = dinv_ref[:, 0:1]
  t = jnp.concatenate([accl_ref[...], accr_ref[...]], axis=1)
  t = jnp.maximum(dinv * t + b_ref[...], 0.0)
  g_ref[...] = dinv * jnp.dot(
      t, w_ref[...], preferred_element_type=jnp.float32
  )


def _tc_final_body(accl_ref, accr_ref, dinv_ref, b_ref, out_ref):
  dinv = dinv_ref[:, 0:1]
  t = dinv * jnp.concatenate([accl_ref[...], accr_ref[...]], axis=1) + b_ref[...]
  m = jnp.max(t, axis=1, keepdims=True)
  u = t - m
  out_ref[...] = u - jnp.log(jnp.sum(jnp.exp(u), axis=1, keepdims=True))


def kernel(x, edge_index, W1, b1, W2, b2, W3, b3):
  n, d = x.shape
  h = W1.shape[1]
  e = edge_index.shape[1]
  hh = h // 2
  rb = 1000
  ng = n // rb
  n_pad = ((n + 16 + 127) // 128) * 128
  e_pad = ((e + _NW * _B - 1) // (_NW * _B)) * (_NW * _B)
  eb_rows = e_pad // _B

  src = edge_index[0]
  dst = edge_index[1]
  pad = e_pad - e
  src2 = jnp.concatenate([src, jnp.zeros((pad,), jnp.int32)]).reshape(eb_rows, _B)
  dst2 = jnp.concatenate(
      [dst, jnp.full((pad,), n, jnp.int32)]
  ).reshape(eb_rows, _B)

  zeros128 = jnp.zeros((n_pad, 128), jnp.float32)
  ones128 = jnp.ones((_B, 128), jnp.float32)

  sc_deg = _make_sc_degree(n_pad, eb_rows, 128)
  sc_agg = _make_sc_aggregate(n, n_pad, eb_rows, hh)

  degp = sc_deg(dst2, zeros128, ones128)
  degsum = (degp[:n_pad, :8] + degp[n_pad:, :8])[:n]

  tc0 = pl.pallas_call(
      _tc0_body,
      grid=(ng, _NC),
      in_specs=[
          pl.BlockSpec((rb, d), lambda i, c: (i, 0)),
          pl.BlockSpec((d, hh), lambda i, c: (0, c)),
          pl.BlockSpec((rb, 8), lambda i, c: (i, 0)),
      ],
      out_specs=[
          pl.BlockSpec((rb, hh), lambda i, c: (c * ng + i, 0)),
          pl.BlockSpec((rb, 8), lambda i, c: (i, 0)),
      ],
      out_shape=[
          jax.ShapeDtypeStruct((_NC * n, hh), jnp.float32),
          jax.ShapeDtypeStruct((n, 8), jnp.float32),
      ],
  )
  g1, dinv8 = tc0(x, W1, degsum)

  def tc_layer(acc, w, b):
    return pl.pallas_call(
        _tc_layer_body,
        grid=(ng, _NC),
        in_specs=[
            pl.BlockSpec((rb, hh), lambda i, c: (i, 0)),
            pl.BlockSpec((rb, hh), lambda i, c: (ng + i, 0)),
            pl.BlockSpec((rb, 8), lambda i, c: (i, 0)),
            pl.BlockSpec((1, h), lambda i, c: (0, 0)),
            pl.BlockSpec((h, hh), lambda i, c: (0, c)),
        ],
        out_specs=pl.BlockSpec((rb, hh), lambda i, c: (c * ng + i, 0)),
        out_shape=jax.ShapeDtypeStruct((_NC * n, hh), jnp.float32),
    )(acc, acc, dinv8, b.reshape(1, h), w)

  acc1 = sc_agg(g1, src2, dst2)
  g2 = tc_layer(acc1, W2, b1)
  acc2 = sc_agg(g2, src2, dst2)
  g3 = tc_layer(acc2, W3, b2)
  acc3 = sc_agg(g3, src2, dst2)

  out = pl.pallas_call(
      _tc_final_body,
      grid=(ng,),
      in_specs=[
          pl.BlockSpec((rb, hh), lambda i: (i, 0)),
          pl.BlockSpec((rb, hh), lambda i: (ng + i, 0)),
          pl.BlockSpec((rb, 8), lambda i: (i, 0)),
          pl.BlockSpec((1, h), lambda i: (0, 0)),
      ],
      out_specs=pl.BlockSpec((rb, h), lambda i: (i, 0)),
      out_shape=jax.ShapeDtypeStruct((n, h), jnp.float32),
  )(acc3, acc3, dinv8, b3.reshape(1, h))
  return out


# speedup vs baseline: 7.2541x; 1.0274x over previous
import functools

import jax
import jax.numpy as jnp
from jax import lax
from jax.experimental import pallas as pl
from jax.experimental.pallas import tpu as pltpu
from jax.experimental.pallas import tpu_sc as plsc

_NC = 2
_NS = 16
_NW = _NC * _NS
_B = 128

_mesh = plsc.VectorSubcoreMesh(
    core_axis_name="c", subcore_axis_name="s", num_cores=_NC, num_subcores=_NS
)


def _make_sc_degree(n_pad, eb_rows, width):
  rows_per_sub = n_pad // _NS
  rows_per_w = eb_rows // _NW

  @functools.partial(
      pl.kernel,
      mesh=_mesh,
      out_type=jax.ShapeDtypeStruct((_NC * n_pad, width), jnp.float32),
      scratch_types=[
          pltpu.VMEM((rows_per_w, _B), jnp.int32),
          pltpu.VMEM((_B, width), jnp.float32),
          pltpu.VMEM_SHARED((n_pad, width), jnp.float32),
      ],
  )
  def deg_kernel(dst_hbm, zeros_hbm, ones_hbm, out_hbm, dst_v, ones_v, deg_sh):
    cid = lax.axis_index("c")
    sid = lax.axis_index("s")
    wid = sid * _NC + cid
    pltpu.sync_copy(
        zeros_hbm.at[pl.ds(sid * rows_per_sub, rows_per_sub)],
        deg_sh.at[pl.ds(sid * rows_per_sub, rows_per_sub)],
    )
    pltpu.sync_copy(ones_hbm, ones_v)
    pltpu.sync_copy(dst_hbm.at[pl.ds(wid * rows_per_w, rows_per_w)], dst_v)
    plsc.subcore_barrier()

    def body(j, carry):
      pltpu.sync_copy(ones_v, deg_sh.at[dst_v.at[j]], add=True)
      return carry

    lax.fori_loop(0, rows_per_w, body, 0)
    plsc.subcore_barrier()
    pltpu.sync_copy(
        deg_sh.at[pl.ds(sid * rows_per_sub, rows_per_sub)],
        out_hbm.at[pl.ds(cid * n_pad + sid * rows_per_sub, rows_per_sub)],
    )

  return deg_kernel


def _make_sc_aggregate(n, n_pad, eb_rows, h_half):
  rows_per_sub = (n // _NS) // 8 * 8
  tail = n - rows_per_sub * _NS
  rows_per_w = eb_rows // _NS

  @functools.partial(
      pl.kernel,
      mesh=_mesh,
      out_type=jax.ShapeDtypeStruct((_NC * n, h_half), jnp.float32),
      scratch_types=[
          pltpu.VMEM((rows_per_w // 2, _B), jnp.int32),
          pltpu.VMEM((rows_per_w // 2, _B), jnp.int32),
          pltpu.VMEM((_B, h_half), jnp.float32),
          pltpu.VMEM((_B, h_half), jnp.float32),
          pltpu.VMEM_SHARED((n_pad, h_half), jnp.float32),
          pltpu.SemaphoreType.DMA,
      ],
  )
  def agg_kernel(g_hbm, src_hbm, dst_hbm, out_hbm, src_v, dst_v, rows_a,
                 rows_b, acc_sh, sem_a):
    cid = lax.axis_index("c")
    sid = lax.axis_index("s")
    pltpu.sync_copy(
        g_hbm.at[pl.ds(cid * n + sid * rows_per_sub, rows_per_sub)],
        acc_sh.at[pl.ds(sid * rows_per_sub, rows_per_sub)],
    )
    if tail:
      @pl.when(sid == 0)
      def _():
        pltpu.sync_copy(
            g_hbm.at[pl.ds(cid * n + rows_per_sub * _NS, tail)],
            acc_sh.at[pl.ds(rows_per_sub * _NS, tail)],
        )
    plsc.subcore_barrier()

    half_rows = rows_per_w // 2
    n_pairs = half_rows // 2

    for chunk in range(2):
      base = sid * rows_per_w + chunk * half_rows
      pltpu.sync_copy(
          src_hbm.at[pl.ds(cid * (_NS * rows_per_w) + base, half_rows)], src_v
      )
      pltpu.sync_copy(dst_hbm.at[pl.ds(base, half_rows)], dst_v)

      pltpu.async_copy(g_hbm.at[src_v.at[0]], rows_a, sem_a)

      def ebody(i, carry):
        j = 2 * i
        pltpu.async_copy(g_hbm.at[src_v.at[j + 1]], rows_b, sem_a)
        pltpu.make_async_copy(g_hbm.at[src_v.at[j]], rows_a, sem_a).wait()
        pltpu.sync_copy(rows_a, acc_sh.at[dst_v.at[j]], add=True)

        @pl.when(i < n_pairs - 1)
        def _():
          pltpu.async_copy(g_hbm.at[src_v.at[j + 2]], rows_a, sem_a)

        pltpu.make_async_copy(g_hbm.at[src_v.at[j + 1]], rows_b, sem_a).wait()
        pltpu.sync_copy(rows_b, acc_sh.at[dst_v.at[j + 1]], add=True)
        return carry

      lax.fori_loop(0, n_pairs, ebody, 0)
    plsc.subcore_barrier()
    pltpu.sync_copy(
        acc_sh.at[pl.ds(sid * rows_per_sub, rows_per_sub)],
        out_hbm.at[pl.ds(cid * n + sid * rows_per_sub, rows_per_sub)],
    )
    if tail:
      @pl.when(sid == 0)
      def _():
        pltpu.sync_copy(
            acc_sh.at[pl.ds(rows_per_sub * _NS, tail)],
            out_hbm.at[pl.ds(cid * n + rows_per_sub * _NS, tail)],
        )

  return agg_kernel


def _tc0_body(x_ref, w_ref, deg_ref, g_ref, dinv_ref):
  dinv = lax.rsqrt(deg_ref[:, 0:1] + 1.0)
  g_ref[...] = dinv * jnp.dot(
      x_ref[...], w_ref[...], preferred_element_type=jnp.float32
  )
  dinv_ref[...] = jnp.broadcast_to(dinv, dinv_ref.shape)


def _tc_layer_body(accl_ref, accr_ref, dinv_ref, b_ref, w_ref, g_ref):
  dinv = dinv_ref[:, 0:1]
  t = jnp.concatenate([accl_ref[...], accr_ref[...]], axis=1)
  t = jnp.maximum(dinv * t + b_ref[...], 0.0)
  g_ref[...] = dinv * jnp.dot(
      t, w_ref[...], preferred_element_type=jnp.float32
  )


def _tc_final_body(accl_ref, accr_ref, dinv_ref, b_ref, out_ref):
  dinv = dinv_ref[:, 0:1]
  t = dinv * jnp.concatenate([accl_ref[...], accr_ref[...]], axis=1) + b_ref[...]
  m = jnp.max(t, axis=1, keepdims=True)
  u = t - m
  out_ref[...] = u - jnp.log(jnp.sum(jnp.exp(u), axis=1, keepdims=True))


def kernel(x, edge_index, W1, b1, W2, b2, W3, b3):
  n, d = x.shape
  h = W1.shape[1]
  e = edge_index.shape[1]
  hh = h // 2
  rb = 1000
  ng = n // rb
  n_pad = ((n + 16 + 127) // 128) * 128
  e_pad = ((e + _NW * _B - 1) // (_NW * _B)) * (_NW * _B)
  eb_rows = e_pad // _B

  src = edge_index[0]
  dst = edge_index[1]
  pad = e_pad - e
  src2 = jnp.concatenate([src, jnp.zeros((pad,), jnp.int32)]).reshape(eb_rows, _B)
  dst2 = jnp.concatenate(
      [dst, jnp.full((pad,), n, jnp.int32)]
  ).reshape(eb_rows, _B)
  src2x2 = jnp.concatenate([src2, src2 + n], axis=0)

  zeros128 = jnp.zeros((n_pad, 128), jnp.float32)
  ones128 = jnp.ones((_B, 128), jnp.float32)

  sc_deg = _make_sc_degree(n_pad, eb_rows, 128)
  sc_agg = _make_sc_aggregate(n, n_pad, eb_rows, hh)

  degp = sc_deg(dst2, zeros128, ones128)
  degsum = (degp[:n_pad, :8] + degp[n_pad:, :8])[:n]

  tc0 = pl.pallas_call(
      _tc0_body,
      grid=(ng, _NC),
      in_specs=[
          pl.BlockSpec((rb, d), lambda i, c: (i, 0)),
          pl.BlockSpec((d, hh), lambda i, c: (0, c)),
          pl.BlockSpec((rb, 8), lambda i, c: (i, 0)),
      ],
      out_specs=[
          pl.BlockSpec((rb, hh), lambda i, c: (c * ng + i, 0)),
          pl.BlockSpec((rb, 8), lambda i, c: (i, 0)),
      ],
      out_shape=[
          jax.ShapeDtypeStruct((_NC * n, hh), jnp.float32),
          jax.ShapeDtypeStruct((n, 8), jnp.float32),
      ],
  )
  g1, dinv8 = tc0(x, W1, degsum)

  def tc_layer(acc, w, b):
    return pl.pallas_call(
        _tc_layer_body,
        grid=(ng, _NC),
        in_specs=[
            pl.BlockSpec((rb, hh), lambda i, c: (i, 0)),
            pl.BlockSpec((rb, hh), lambda i, c: (ng + i, 0)),
            pl.BlockSpec((rb, 8), lambda i, c: (i, 0)),
            pl.BlockSpec((1, h), lambda i, c: (0, 0)),
            pl.BlockSpec((h, hh), lambda i, c: (0, c)),
        ],
        out_specs=pl.BlockSpec((rb, hh), lambda i, c: (c * ng + i, 0)),
        out_shape=jax.ShapeDtypeStruct((_NC * n, hh), jnp.float32),
    )(acc, acc, dinv8, b.reshape(1, h), w)

  acc1 = sc_agg(g1, src2x2, dst2)
  g2 = tc_layer(acc1, W2, b1)
  acc2 = sc_agg(g2, src2x2, dst2)
  g3 = tc_layer(acc2, W3, b2)
  acc3 = sc_agg(g3, src2x2, dst2)

  out = pl.pallas_call(
      _tc_final_body,
      grid=(ng,),
      in_specs=[
          pl.BlockSpec((rb, hh), lambda i: (i, 0)),
          pl.BlockSpec((rb, hh), lambda i: (ng + i, 0)),
          pl.BlockSpec((rb, 8), lambda i: (i, 0)),
          pl.BlockSpec((1, h), lambda i: (0, 0)),
      ],
      out_specs=pl.BlockSpec((rb, h), lambda i: (i, 0)),
      out_shape=jax.ShapeDtypeStruct((n, h), jnp.float32),
  )(acc3, acc3, dinv8, b3.reshape(1, h))
  return out
